# speedup vs baseline: 1.5462x; 1.5462x over previous
import jax
import jax.numpy as jnp
from jax.experimental import pallas as pl

_BS = 512


def _add_kernel(x_ref, pe_ref, o_ref):
    o_ref[...] = x_ref[...] + pe_ref[...][None, :, :]


def kernel(x, pos_embed):
    batch, seq_len, d_model = x.shape
    grid = (seq_len // _BS, batch)
    return pl.pallas_call(
        _add_kernel,
        grid=grid,
        in_specs=[
            pl.BlockSpec((1, _BS, d_model), lambda s, b: (b, s, 0)),
            pl.BlockSpec((_BS, d_model), lambda s, b: (s, 0)),
        ],
        out_specs=pl.BlockSpec((1, _BS, d_model), lambda s, b: (b, s, 0)),
        out_shape=jax.ShapeDtypeStruct(x.shape, x.dtype),
    )(x, pos_embed)


# speedup vs baseline: 1.5934x; 1.0306x over previous
import jax
import jax.numpy as jnp
from jax.experimental import pallas as pl

_BS = 1024


def _add_kernel(x_ref, pe_ref, o_ref):
    o_ref[...] = x_ref[...] + pe_ref[...][None, :, :]


def kernel(x, pos_embed):
    batch, seq_len, d_model = x.shape
    grid = (seq_len // _BS, batch)
    return pl.pallas_call(
        _add_kernel,
        grid=grid,
        in_specs=[
            pl.BlockSpec((1, _BS, d_model), lambda s, b: (b, s, 0)),
            pl.BlockSpec((_BS, d_model), lambda s, b: (s, 0)),
        ],
        out_specs=pl.BlockSpec((1, _BS, d_model), lambda s, b: (b, s, 0)),
        out_shape=jax.ShapeDtypeStruct(x.shape, x.dtype),
    )(x, pos_embed)
